# Initial kernel scaffold; baseline (speedup 1.0000x reference)
import jax
import jax.numpy as jnp
from jax.experimental import pallas as pl


def kernel(text, emb_table, fc_w, fc_b):
    raise NotImplementedError("write your pallas kernel here")


# speedup vs baseline: 48.7514x; 48.7514x over previous
import functools

import jax
import jax.numpy as jnp
from jax import lax
from jax.experimental import pallas as pl
from jax.experimental.pallas import tpu as pltpu
from jax.experimental.pallas import tpu_sc as plsc

VOCAB = 25000
EMB_DIM = 100
SEQ_LEN = 200
BATCH = 4096
POOL_K = 5
T_OUT = SEQ_LEN // POOL_K

NUM_CORES = 2
NUM_SUBCORES = 16
LANES = 16
NW = NUM_CORES * NUM_SUBCORES
B_PER_W = BATCH // NW
NCHUNK = B_PER_W // LANES
OUT_PER_W = B_PER_W * T_OUT

VBLK = 1000


def _scores_body(emb_ref, w_ref, b_ref, out_ref):
    prod = lax.dot_general(
        w_ref[...], emb_ref[...],
        dimension_numbers=(((1,), (1,)), ((), ())),
        preferred_element_type=jnp.float32,
    )
    out_ref[0] = prod * (1.0 / POOL_K) + b_ref[0, 0] * (1.0 / POOL_K)


def _compute_scores(emb_table, fc_w, fc_b):
    nblk = VOCAB // VBLK
    out = pl.pallas_call(
        _scores_body,
        grid=(nblk,),
        in_specs=[
            pl.BlockSpec((VBLK, EMB_DIM), lambda i: (i, 0)),
            pl.BlockSpec((1, EMB_DIM), lambda i: (0, 0)),
            pl.BlockSpec((1, 1), lambda i: (0, 0)),
        ],
        out_specs=pl.BlockSpec((1, 1, VBLK), lambda i: (i, 0, 0)),
        out_shape=jax.ShapeDtypeStruct((nblk, 1, VBLK), jnp.float32),
    )(emb_table, fc_w, fc_b.reshape(1, 1))
    return out.reshape(VOCAB)


@functools.partial(
    pl.kernel,
    mesh=plsc.VectorSubcoreMesh(core_axis_name="c", subcore_axis_name="s"),
    out_type=jax.ShapeDtypeStruct((BATCH * T_OUT,), jnp.float32),
    compiler_params=pltpu.CompilerParams(needs_layout_passes=False),
    scratch_types=[
        pltpu.VMEM((VOCAB,), jnp.float32),
        pltpu.VMEM((SEQ_LEN, B_PER_W), jnp.int32),
        pltpu.VMEM((OUT_PER_W,), jnp.float32),
    ],
)
def _sc_pool(scores_hbm, text_hbm, out_hbm, scores_v, text_v, out_v):
    wid = lax.axis_index("s") * NUM_CORES + lax.axis_index("c")
    pltpu.sync_copy(scores_hbm, scores_v)
    pltpu.sync_copy(text_hbm.at[wid], text_v)

    lane = lax.iota(jnp.int32, LANES)

    def t_body(t, carry):
        s0 = t * POOL_K
        for c in range(NCHUNK):
            acc = plsc.load_gather(scores_v, [text_v[s0, pl.ds(c * LANES, LANES)]])
            for k in range(1, POOL_K):
                idx = text_v[s0 + k, pl.ds(c * LANES, LANES)]
                acc = acc + plsc.load_gather(scores_v, [idx])
            oidx = lane * T_OUT + (c * LANES * T_OUT + t)
            plsc.store_scatter(out_v, [oidx], acc)
        return carry

    lax.fori_loop(0, T_OUT, t_body, 0)
    pltpu.sync_copy(out_v, out_hbm.at[pl.ds(wid * OUT_PER_W, OUT_PER_W)])


def kernel(text, emb_table, fc_w, fc_b):
    scores = _compute_scores(emb_table, fc_w, fc_b)
    text_r = text.reshape(SEQ_LEN, NW, B_PER_W).transpose(1, 0, 2)
    out_flat = _sc_pool(scores, text_r)
    return out_flat.reshape(BATCH, T_OUT, 1)


# speedup vs baseline: 60.1262x; 1.2333x over previous
import functools

import jax
import jax.numpy as jnp
from jax import lax
from jax.experimental import pallas as pl
from jax.experimental.pallas import tpu as pltpu
from jax.experimental.pallas import tpu_sc as plsc

VOCAB = 25000
EMB_DIM = 100
SEQ_LEN = 200
BATCH = 4096
POOL_K = 5
T_OUT = SEQ_LEN // POOL_K

NUM_CORES = 2
NUM_SUBCORES = 16
LANES = 16
NW = NUM_CORES * NUM_SUBCORES
B_PER_W = BATCH // NW
NCHUNK = B_PER_W // LANES
OUT_PER_W = B_PER_W * T_OUT

VBLK = 5000


def _scores_body(emb_ref, w_ref, b_ref, out_ref):
    prod = lax.dot_general(
        w_ref[...], emb_ref[...],
        dimension_numbers=(((1,), (1,)), ((), ())),
        preferred_element_type=jnp.float32,
    )
    out_ref[0] = prod * (1.0 / POOL_K) + b_ref[0, 0] * (1.0 / POOL_K)


def _compute_scores(emb_table, fc_w, fc_b):
    nblk = VOCAB // VBLK
    out = pl.pallas_call(
        _scores_body,
        grid=(nblk,),
        in_specs=[
            pl.BlockSpec((VBLK, EMB_DIM), lambda i: (i, 0)),
            pl.BlockSpec((1, EMB_DIM), lambda i: (0, 0)),
            pl.BlockSpec((1, 1), lambda i: (0, 0)),
        ],
        out_specs=pl.BlockSpec((1, 1, VBLK), lambda i: (i, 0, 0)),
        out_shape=jax.ShapeDtypeStruct((nblk, 1, VBLK), jnp.float32),
    )(emb_table, fc_w, fc_b.reshape(1, 1))
    return out.reshape(VOCAB)


@functools.partial(
    pl.kernel,
    mesh=plsc.VectorSubcoreMesh(core_axis_name="c", subcore_axis_name="s"),
    out_type=jax.ShapeDtypeStruct((BATCH * T_OUT,), jnp.float32),
    compiler_params=pltpu.CompilerParams(needs_layout_passes=False),
    scratch_types=[
        pltpu.VMEM((VOCAB,), jnp.float32),
        pltpu.VMEM((SEQ_LEN, B_PER_W), jnp.int32),
        pltpu.VMEM((OUT_PER_W,), jnp.float32),
        pltpu.SemaphoreType.DMA,
        pltpu.SemaphoreType.DMA,
    ],
)
def _sc_pool(scores_hbm, text_hbm, out_hbm, scores_v, text_v, out_v, sem_s, sem_t):
    wid = lax.axis_index("s") * NUM_CORES + lax.axis_index("c")
    cp_s = pltpu.async_copy(scores_hbm, scores_v, sem_s)
    cp_t = pltpu.async_copy(
        text_hbm.at[:, pl.ds(wid * B_PER_W, B_PER_W)], text_v, sem_t)
    cp_s.wait()
    cp_t.wait()

    lane = lax.iota(jnp.int32, LANES)

    def t_body(t, carry):
        s0 = t * POOL_K
        for c in range(NCHUNK):
            acc = plsc.load_gather(scores_v, [text_v[s0, pl.ds(c * LANES, LANES)]])
            for k in range(1, POOL_K):
                idx = text_v[s0 + k, pl.ds(c * LANES, LANES)]
                acc = acc + plsc.load_gather(scores_v, [idx])
            oidx = lane * T_OUT + (c * LANES * T_OUT + t)
            plsc.store_scatter(out_v, [oidx], acc)
        return carry

    lax.fori_loop(0, T_OUT, t_body, 0)
    pltpu.sync_copy(out_v, out_hbm.at[pl.ds(wid * OUT_PER_W, OUT_PER_W)])


def kernel(text, emb_table, fc_w, fc_b):
    scores = _compute_scores(emb_table, fc_w, fc_b)
    out_flat = _sc_pool(scores, text)
    return out_flat.reshape(BATCH, T_OUT, 1)
